# baseline (speedup 1.0000x reference)
import functools

import jax
import jax.numpy as jnp
from jax import lax
from jax.experimental import pallas as pl
from jax.experimental.pallas import tpu as pltpu
from jax.experimental.pallas import tpu_sc as plsc

B = 16384
D = 32
L = 16

try:
    _info = plsc.get_sparse_core_info()
    NC, NS = _info.num_cores, _info.num_subcores
except ValueError:
    NC, NS = 2, 16
NW = NC * NS
BPW = B // NW
CH = 128
NCH = BPW // CH

_mesh = plsc.VectorSubcoreMesh(core_axis_name="c", subcore_axis_name="s")


@functools.partial(
    pl.kernel,
    mesh=_mesh,
    out_type=jax.ShapeDtypeStruct((B,), jnp.float32),
    compiler_params=pltpu.CompilerParams(
        needs_layout_passes=False, use_tc_tiling_on_sc=False
    ),
    scratch_types=[
        pltpu.VMEM((NCH, CH), jnp.int32),
        pltpu.VMEM((NCH, CH), jnp.int32),
        pltpu.VMEM((BPW, D), jnp.float32),
        pltpu.VMEM((BPW, D), jnp.float32),
        pltpu.VMEM((BPW,), jnp.float32),
        pltpu.VMEM((BPW,), jnp.float32),
        pltpu.VMEM((BPW,), jnp.float32),
        pltpu.SemaphoreType.DMA,
    ],
)
def _mf_sc(users_hbm, items_hbm, ue_hbm, ie_hbm, bu_hbm, bi_hbm, out_hbm,
           uidx, iidx, urows, irows, ubias, ibias, outv, sem):
    wid = lax.axis_index("s") * NC + lax.axis_index("c")
    base = wid * BPW

    for j in range(NCH):
        pltpu.sync_copy(users_hbm.at[pl.ds(base + j * CH, CH)], uidx.at[j])
        pltpu.sync_copy(items_hbm.at[pl.ds(base + j * CH, CH)], iidx.at[j])

    copies = []
    for j in range(NCH):
        sl = pl.ds(j * CH, CH)
        copies.append(pltpu.async_copy(ue_hbm.at[uidx.at[j]], urows.at[sl], sem))
        copies.append(pltpu.async_copy(ie_hbm.at[iidx.at[j]], irows.at[sl], sem))
        copies.append(pltpu.async_copy(bu_hbm.at[uidx.at[j]], ubias.at[sl], sem))
        copies.append(pltpu.async_copy(bi_hbm.at[iidx.at[j]], ibias.at[sl], sem))
    for c in copies:
        c.wait()

    iota = lax.iota(jnp.int32, L)

    def group(g, carry):
        rows16 = g * L + iota
        acc = ubias[pl.ds(g * L, L)] + ibias[pl.ds(g * L, L)]
        for d in range(D):
            col = jnp.full((L,), d, jnp.int32)
            uv = plsc.load_gather(urows, [rows16, col])
            iv = plsc.load_gather(irows, [rows16, col])
            acc = acc + uv * iv
        outv[pl.ds(g * L, L)] = acc
        return carry

    lax.fori_loop(0, BPW // L, group, 0)

    pltpu.sync_copy(outv, out_hbm.at[pl.ds(base, BPW)])


def kernel(users, items, user_embedding, item_embedding, user_biases, item_biases):
    return _mf_sc(
        users.astype(jnp.int32),
        items.astype(jnp.int32),
        user_embedding,
        item_embedding,
        user_biases.reshape(-1),
        item_biases.reshape(-1),
    )
